# baseline (speedup 1.0000x reference)
import functools

import jax
import jax.numpy as jnp
from jax import lax
from jax.experimental import pallas as pl
from jax.experimental.pallas import tpu as pltpu
from jax.experimental.pallas import tpu_sc as plsc

EPS_ = 1e-07
MAX_NORM_ = 1 - 0.0001

NUM_WORKERS = 32
ROWS_PER_CHUNK = 128
D = 16
LANE = 128


def _project_chunk(gbuf, sbuf):
    lane = lax.iota(jnp.int32, 16)

    def block(b, carry):
        f = lane + b * 16
        diags = []
        ssum = jnp.zeros((16,), jnp.float32)
        for j in range(D):
            d2 = (lane + j) & (D - 1)
            dg = plsc.load_gather(gbuf, [f, d2])
            diags.append(dg)
            ssum = ssum + dg * dg
        bits = lax.bitcast_convert_type(ssum, jnp.int32)
        y = lax.bitcast_convert_type(
            jnp.int32(0x5F3759DF) - (bits >> 1), jnp.float32)
        for _ in range(3):
            y = y * (1.5 - 0.5 * ssum * y * y)
        norm = ssum * y
        factor = jnp.where(norm >= MAX_NORM_, MAX_NORM_ * y,
                           jnp.ones((16,), jnp.float32))
        for j in range(D):
            d2 = (lane + j) & (D - 1)
            p = f * D + d2
            plsc.store_scatter(sbuf, [p >> 7, p & (LANE - 1)],
                               diags[j] * factor)
        return carry

    lax.fori_loop(0, ROWS_PER_CHUNK // 16, block, 0)


def _make_sc_kernel(n_flat):
    rows_per_worker = n_flat // NUM_WORKERS
    chunks = rows_per_worker // ROWS_PER_CHUNK
    out_rows_per_chunk = ROWS_PER_CHUNK * D // LANE
    info = plsc.get_sparse_core_info()
    nc = info.num_cores
    mesh = plsc.VectorSubcoreMesh(core_axis_name="c", subcore_axis_name="s")
    gbuf_t = pltpu.VMEM((ROWS_PER_CHUNK, D), jnp.float32)
    sbuf_t = pltpu.VMEM((out_rows_per_chunk, LANE), jnp.float32)

    @functools.partial(
        pl.kernel,
        mesh=mesh,
        out_type=jax.ShapeDtypeStruct((n_flat * D // LANE, LANE),
                                      jnp.float32),
        compiler_params=pltpu.CompilerParams(needs_layout_passes=False,
                                             use_tc_tiling_on_sc=False),
        scratch_types=[
            pltpu.VMEM((chunks, ROWS_PER_CHUNK), jnp.int32),
            gbuf_t,
            gbuf_t,
            sbuf_t,
            sbuf_t,
            pltpu.SemaphoreType.DMA,
            pltpu.SemaphoreType.DMA,
            pltpu.SemaphoreType.DMA,
            pltpu.SemaphoreType.DMA,
        ],
    )
    def sc_kernel(idx_hbm, emb_hbm, out_hbm, idx_v, ga, gb, sa, sb, gsem_a,
                  gsem_b, ssem_a, ssem_b):
        wid = lax.axis_index("s") * nc + lax.axis_index("c")
        idx_row0 = wid * chunks
        out_row0 = wid * chunks * out_rows_per_chunk
        pltpu.sync_copy(idx_hbm.at[pl.ds(idx_row0, chunks)], idx_v)

        def gather_to(c, buf, gsem):
            return pltpu.make_async_copy(emb_hbm.at[idx_v.at[c]], buf, gsem)

        def store_of(c, buf, ssem):
            return pltpu.make_async_copy(
                buf,
                out_hbm.at[pl.ds(out_row0 + c * out_rows_per_chunk,
                                 out_rows_per_chunk)],
                ssem)

        gather_to(0, ga, gsem_a).start()

        def halfstep(c, gbuf, gsem, sbuf, ssem, ngbuf, ngsem):
            gather_to(c, gbuf, gsem).wait()

            @pl.when(c + 1 < chunks)
            def _():
                gather_to(c + 1, ngbuf, ngsem).start()

            @pl.when(c >= 2)
            def _():
                store_of(c - 2, sbuf, ssem).wait()

            _project_chunk(gbuf, sbuf)
            store_of(c, sbuf, ssem).start()

        def step(t, carry):
            halfstep(2 * t, ga, gsem_a, sa, ssem_a, gb, gsem_b)
            halfstep(2 * t + 1, gb, gsem_b, sb, ssem_b, ga, gsem_a)
            return carry

        lax.fori_loop(0, chunks // 2, step, 0)
        store_of(chunks - 2, sa, ssem_a).wait()
        store_of(chunks - 1, sb, ssem_b).wait()

    return sc_kernel


def kernel(idx, emb):
    n_idx, seq_len = idx.shape
    n_flat = n_idx * seq_len
    idx2 = idx.astype(jnp.int32).reshape(n_flat // ROWS_PER_CHUNK,
                                         ROWS_PER_CHUNK)
    out2 = _make_sc_kernel(n_flat)(idx2, emb)
    return out2.reshape(n_idx, seq_len, D)


# speedup vs baseline: 1.1141x; 1.1141x over previous
import functools

import jax
import jax.numpy as jnp
from jax import lax
from jax.experimental import pallas as pl
from jax.experimental.pallas import tpu as pltpu
from jax.experimental.pallas import tpu_sc as plsc

EPS_ = 1e-07
MAX_NORM_ = 1 - 0.0001

NUM_WORKERS = 32
IDX_ROWS_PER_CHUNK = 8
D = 16
LANE = 128


def _project_chunk(gbuf, sbuf, n_rows, seq_len):
    lane = lax.iota(jnp.int32, 16)

    def block(b, carry):
        f = lane + b * 16
        d0 = f // seq_len
        d1 = f % seq_len
        diags = []
        ssum = jnp.zeros((16,), jnp.float32)
        for j in range(D):
            d2 = (lane + j) & (D - 1)
            dg = plsc.load_gather(gbuf, [d0, d1, d2])
            diags.append(dg)
            ssum = ssum + dg * dg
        bits = lax.bitcast_convert_type(ssum, jnp.int32)
        y = lax.bitcast_convert_type(
            jnp.int32(0x5F3759DF) - (bits >> 1), jnp.float32)
        for _ in range(3):
            y = y * (1.5 - 0.5 * ssum * y * y)
        norm = ssum * y
        factor = jnp.where(norm >= MAX_NORM_, MAX_NORM_ * y,
                           jnp.ones((16,), jnp.float32))
        for j in range(D):
            d2 = (lane + j) & (D - 1)
            plsc.store_scatter(sbuf, [d0 * D + d2, d1], diags[j] * factor)
        return carry

    lax.fori_loop(0, n_rows // 16, block, 0)


def _make_sc_kernel(n_idx, seq_len):
    idx_rows_per_worker = n_idx // NUM_WORKERS
    chunks = idx_rows_per_worker // IDX_ROWS_PER_CHUNK
    rows_per_chunk = IDX_ROWS_PER_CHUNK * seq_len
    out_rows_per_chunk = IDX_ROWS_PER_CHUNK * D
    info = plsc.get_sparse_core_info()
    nc = info.num_cores
    mesh = plsc.VectorSubcoreMesh(core_axis_name="c", subcore_axis_name="s")
    gbuf_t = pltpu.VMEM((IDX_ROWS_PER_CHUNK, seq_len, D), jnp.float32)
    sbuf_t = pltpu.VMEM((out_rows_per_chunk, LANE), jnp.float32)

    @functools.partial(
        pl.kernel,
        mesh=mesh,
        out_type=jax.ShapeDtypeStruct((n_idx * D, LANE), jnp.float32),
        compiler_params=pltpu.CompilerParams(needs_layout_passes=False,
                                             use_tc_tiling_on_sc=False),
        scratch_types=[
            pltpu.VMEM((idx_rows_per_worker, seq_len), jnp.int32),
            gbuf_t,
            gbuf_t,
            sbuf_t,
            sbuf_t,
            pltpu.SemaphoreType.DMA,
            pltpu.SemaphoreType.DMA,
            pltpu.SemaphoreType.DMA,
            pltpu.SemaphoreType.DMA,
        ],
    )
    def sc_kernel(idx_hbm, emb_hbm, out_hbm, idx_v, ga, gb, sa, sb, gsem_a,
                  gsem_b, ssem_a, ssem_b):
        wid = lax.axis_index("s") * nc + lax.axis_index("c")
        base = wid * idx_rows_per_worker
        out_row0 = base * D
        pltpu.sync_copy(idx_hbm.at[pl.ds(base, idx_rows_per_worker)], idx_v)

        lane = lax.iota(jnp.int32, 16)
        zeros16 = jnp.zeros((16,), jnp.float32)

        def zrow(r, carry):
            for k in range(LANE // 16):
                plsc.store_scatter(sa, [jnp.full((16,), r, jnp.int32),
                                        lane + k * 16], zeros16)
                plsc.store_scatter(sb, [jnp.full((16,), r, jnp.int32),
                                        lane + k * 16], zeros16)
            return carry

        lax.fori_loop(0, out_rows_per_chunk, zrow, 0)

        def start_gathers(c, buf, gsem):
            r0 = c * IDX_ROWS_PER_CHUNK
            for k in range(IDX_ROWS_PER_CHUNK):
                pltpu.make_async_copy(
                    emb_hbm.at[idx_v.at[r0 + k]], buf.at[k], gsem).start()

        def wait_gathers(c, buf, gsem):
            r0 = c * IDX_ROWS_PER_CHUNK
            for k in range(IDX_ROWS_PER_CHUNK):
                pltpu.make_async_copy(
                    emb_hbm.at[idx_v.at[r0 + k]], buf.at[k], gsem).wait()

        def store_of(c, buf, ssem):
            return pltpu.make_async_copy(
                buf,
                out_hbm.at[pl.ds(out_row0 + c * out_rows_per_chunk,
                                 out_rows_per_chunk)],
                ssem)

        start_gathers(0, ga, gsem_a)

        def halfstep(c, gbuf, gsem, sbuf, ssem, ngbuf, ngsem):
            wait_gathers(c, gbuf, gsem)

            @pl.when(c + 1 < chunks)
            def _():
                start_gathers(c + 1, ngbuf, ngsem)

            @pl.when(c >= 2)
            def _():
                store_of(c - 2, sbuf, ssem).wait()

            _project_chunk(gbuf, sbuf, rows_per_chunk, seq_len)
            store_of(c, sbuf, ssem).start()

        def step(t, carry):
            halfstep(2 * t, ga, gsem_a, sa, ssem_a, gb, gsem_b)
            halfstep(2 * t + 1, gb, gsem_b, sb, ssem_b, ga, gsem_a)
            return carry

        lax.fori_loop(0, chunks // 2, step, 0)
        store_of(chunks - 2, sa, ssem_a).wait()
        store_of(chunks - 1, sb, ssem_b).wait()

    return sc_kernel


def kernel(idx, emb):
    n_idx, seq_len = idx.shape
    out2 = _make_sc_kernel(n_idx, seq_len)(idx.astype(jnp.int32), emb)
    out3 = out2.reshape(n_idx, D, LANE)[:, :, :seq_len]
    return out3.transpose(0, 2, 1)


# speedup vs baseline: 1.1172x; 1.0029x over previous
import functools

import jax
import jax.numpy as jnp
from jax import lax
from jax.experimental import pallas as pl
from jax.experimental.pallas import tpu as pltpu
from jax.experimental.pallas import tpu_sc as plsc

EPS_ = 1e-07
MAX_NORM_ = 1 - 0.0001

NUM_WORKERS = 32
IDX_ROWS_PER_CHUNK = 8
D = 16
LANE = 128


def _project_chunk(gbuf, sbuf, n_rows, seq_len):
    lane = lax.iota(jnp.int32, 16)

    def block(b, carry):
        f = lane + b * 16
        d0 = f // seq_len
        d1 = f % seq_len
        diags = []
        ssum = jnp.zeros((16,), jnp.float32)
        for j in range(D):
            d2 = (lane + j) & (D - 1)
            dg = plsc.load_gather(gbuf, [d0, d1, d2])
            diags.append(dg)
            ssum = ssum + dg * dg
        bits = lax.bitcast_convert_type(ssum, jnp.int32)
        y = lax.bitcast_convert_type(
            jnp.int32(0x5F3759DF) - (bits >> 1), jnp.float32)
        for _ in range(3):
            y = y * (1.5 - 0.5 * ssum * y * y)
        norm = ssum * y
        factor = jnp.where(norm >= MAX_NORM_, MAX_NORM_ * y,
                           jnp.ones((16,), jnp.float32))
        for j in range(D):
            d2 = (lane + j) & (D - 1)
            plsc.store_scatter(sbuf, [d0 * D + d2, d1], diags[j] * factor)
        return carry

    lax.fori_loop(0, n_rows // 16, block, 0)


def _make_sc_kernel(n_idx, seq_len):
    idx_rows_per_worker = n_idx // NUM_WORKERS
    chunks = idx_rows_per_worker // IDX_ROWS_PER_CHUNK
    rows_per_chunk = IDX_ROWS_PER_CHUNK * seq_len
    out_rows_per_chunk = IDX_ROWS_PER_CHUNK * D
    info = plsc.get_sparse_core_info()
    nc = info.num_cores
    mesh = plsc.VectorSubcoreMesh(core_axis_name="c", subcore_axis_name="s")
    gbuf_t = pltpu.VMEM((IDX_ROWS_PER_CHUNK, seq_len, D), jnp.float32)
    sbuf_t = pltpu.VMEM((out_rows_per_chunk, LANE), jnp.float32)

    @functools.partial(
        pl.kernel,
        mesh=mesh,
        out_type=jax.ShapeDtypeStruct((n_idx * D, LANE), jnp.float32),
        compiler_params=pltpu.CompilerParams(needs_layout_passes=False,
                                             use_tc_tiling_on_sc=False),
        scratch_types=[
            pltpu.VMEM((seq_len, idx_rows_per_worker), jnp.int32),
            pltpu.VMEM((idx_rows_per_worker, seq_len), jnp.int32),
            gbuf_t,
            gbuf_t,
            sbuf_t,
            sbuf_t,
            pltpu.SemaphoreType.DMA,
            pltpu.SemaphoreType.DMA,
            pltpu.SemaphoreType.DMA,
            pltpu.SemaphoreType.DMA,
        ],
    )
    def sc_kernel(idx_hbm, emb_hbm, out_hbm, idx_tv, idx_v, ga, gb, sa, sb,
                  gsem_a, gsem_b, ssem_a, ssem_b):
        wid = lax.axis_index("s") * nc + lax.axis_index("c")
        base = wid * idx_rows_per_worker
        out_row0 = base * D
        pltpu.sync_copy(idx_hbm.at[:, pl.ds(base, idx_rows_per_worker)],
                        idx_tv)
        lane = lax.iota(jnp.int32, 16)
        zeros16 = jnp.zeros((16,), jnp.float32)

        def repack(c, carry):
            col = c * 16 + lane
            for j in range(seq_len):
                jj = jnp.full((16,), j, jnp.int32)
                v = plsc.load_gather(idx_tv, [jj, col])
                plsc.store_scatter(idx_v, [col, jj], v)
            return carry

        lax.fori_loop(0, idx_rows_per_worker // 16, repack, 0)


        def zrow(r, carry):
            for k in range(LANE // 16):
                plsc.store_scatter(sa, [jnp.full((16,), r, jnp.int32),
                                        lane + k * 16], zeros16)
                plsc.store_scatter(sb, [jnp.full((16,), r, jnp.int32),
                                        lane + k * 16], zeros16)
            return carry

        lax.fori_loop(0, out_rows_per_chunk, zrow, 0)

        def start_gathers(c, buf, gsem):
            r0 = c * IDX_ROWS_PER_CHUNK
            for k in range(IDX_ROWS_PER_CHUNK):
                pltpu.make_async_copy(
                    emb_hbm.at[idx_v.at[r0 + k]], buf.at[k], gsem).start()

        def wait_gathers(c, buf, gsem):
            r0 = c * IDX_ROWS_PER_CHUNK
            for k in range(IDX_ROWS_PER_CHUNK):
                pltpu.make_async_copy(
                    emb_hbm.at[idx_v.at[r0 + k]], buf.at[k], gsem).wait()

        def store_of(c, buf, ssem):
            return pltpu.make_async_copy(
                buf,
                out_hbm.at[pl.ds(out_row0 + c * out_rows_per_chunk,
                                 out_rows_per_chunk)],
                ssem)

        start_gathers(0, ga, gsem_a)

        def halfstep(c, gbuf, gsem, sbuf, ssem, ngbuf, ngsem):
            wait_gathers(c, gbuf, gsem)

            @pl.when(c + 1 < chunks)
            def _():
                start_gathers(c + 1, ngbuf, ngsem)

            @pl.when(c >= 2)
            def _():
                store_of(c - 2, sbuf, ssem).wait()

            _project_chunk(gbuf, sbuf, rows_per_chunk, seq_len)
            store_of(c, sbuf, ssem).start()

        def step(t, carry):
            halfstep(2 * t, ga, gsem_a, sa, ssem_a, gb, gsem_b)
            halfstep(2 * t + 1, gb, gsem_b, sb, ssem_b, ga, gsem_a)
            return carry

        lax.fori_loop(0, chunks // 2, step, 0)
        store_of(chunks - 2, sa, ssem_a).wait()
        store_of(chunks - 1, sb, ssem_b).wait()

    return sc_kernel


def kernel(idx, emb):
    n_idx, seq_len = idx.shape
    out2 = _make_sc_kernel(n_idx, seq_len)(idx.astype(jnp.int32).T, emb)
    out3 = out2.reshape(n_idx, D, LANE)[:, :, :seq_len]
    return out3.transpose(0, 2, 1)
